# baseline (speedup 1.0000x reference)
import functools

import jax
import jax.numpy as jnp
from jax import lax
from jax.experimental import pallas as pl
from jax.experimental.pallas import tpu as pltpu
from jax.experimental.pallas import tpu_sc as plsc

N = 10000
E = 160000
G = 64
OUT = 2048

_B = 128
_NB = E // _B
_NBW = 40
_NTILES = 16
_STRIPE = 624
_TAIL = N - _NTILES * _STRIPE
_R = 400
_NR = N // _R

_HI = jax.lax.Precision.HIGHEST


@functools.lru_cache(maxsize=None)
def _make_sc_segsum(C):
    mesh = plsc.VectorSubcoreMesh(core_axis_name="c", subcore_axis_name="s",
                                  num_cores=2, num_subcores=_NTILES)

    @functools.partial(
        pl.kernel,
        out_type=jax.ShapeDtypeStruct((2 * C * N, 128), jnp.float32),
        mesh=mesh,
        scratch_types=[
            pltpu.VMEM((_NBW, _B), jnp.int32),
            pltpu.VMEM((_NBW, _B), jnp.int32),
            pltpu.VMEM((_B, 128), jnp.float32),
            pltpu.VMEM((_B, 128), jnp.float32),
            pltpu.VMEM((24, 128), jnp.float32),
            pltpu.VMEM_SHARED((N, 128), jnp.float32),
            pltpu.SemaphoreType.DMA,
            pltpu.SemaphoreType.DMA,
            pltpu.SemaphoreType.DMA,
            pltpu.SemaphoreType.DMA,
        ],
    )
    def sc_kernel(ych, src_hbm, dst_hbm, out_hbm, sidx, didx, rows0, rows1,
                  zbuf, table, sem0, sem1, ssem0, ssem1):
        cid = lax.axis_index("c")
        sid = lax.axis_index("s")
        w = cid * _NTILES + sid
        nb = jnp.where(w < _NB - (_NBW - 1) * 32, _NBW, _NBW - 1)

        @pl.loop(0, 24)
        def _(r):
            @pl.loop(0, 128, step=16)
            def _(j):
                zbuf.at[r, pl.ds(j, 16)][...] = jnp.zeros((16,), jnp.float32)

        pltpu.sync_copy(dst_hbm.at[pl.ds(w * _NBW, _NBW)], didx)

        for c in range(C):
            @pl.loop(0, _STRIPE // 24)
            def _(q):
                pltpu.sync_copy(
                    zbuf.at[pl.ds(0, 24)],
                    table.at[pl.ds(sid * _STRIPE + q * 24, 24)])

            @pl.when(sid == _NTILES - 1)
            def _():
                pltpu.sync_copy(zbuf.at[pl.ds(0, _TAIL)],
                                table.at[pl.ds(_NTILES * _STRIPE, _TAIL)])

            pltpu.sync_copy(src_hbm.at[pl.ds((c * 32 + w) * _NBW, _NBW)],
                            sidx)
            plsc.subcore_barrier()

            pltpu.async_copy(ych.at[sidx.at[0]], rows0, sem0)
            pltpu.async_copy(ych.at[sidx.at[1]], rows1, sem1)

            @pl.loop(0, _NBW // 2)
            def _(p):
                i0 = 2 * p
                i1 = i0 + 1
                pltpu.make_async_copy(ych.at[sidx.at[i0]], rows0, sem0).wait()
                pltpu.async_copy(rows0, table.at[didx.at[i0]], ssem0,
                                 add=True)

                @pl.when(i1 < nb)
                def _():
                    pltpu.make_async_copy(ych.at[sidx.at[i1]], rows1,
                                          sem1).wait()
                    pltpu.async_copy(rows1, table.at[didx.at[i1]], ssem1,
                                     add=True)

                @pl.when(i0 + 2 < nb)
                def _():
                    pltpu.make_async_copy(
                        rows0, table.at[didx.at[i0]], ssem0).wait()
                    pltpu.async_copy(ych.at[sidx.at[i0 + 2]], rows0, sem0)

                @pl.when(i1 + 2 < nb)
                def _():
                    pltpu.make_async_copy(
                        rows1, table.at[didx.at[i1]], ssem1).wait()
                    pltpu.async_copy(ych.at[sidx.at[i1 + 2]], rows1, sem1)
            pltpu.make_async_copy(rows0, table.at[didx.at[0]], ssem0).wait()
            pltpu.make_async_copy(rows1, table.at[didx.at[0]], ssem1).wait()
            plsc.subcore_barrier()

            base = (cid * C + c) * N
            pltpu.sync_copy(table.at[pl.ds(sid * _STRIPE, _STRIPE)],
                            out_hbm.at[pl.ds(base + sid * _STRIPE, _STRIPE)])

            @pl.when(sid == _NTILES - 1)
            def _():
                pltpu.sync_copy(
                    table.at[pl.ds(_NTILES * _STRIPE, _TAIL)],
                    out_hbm.at[pl.ds(base + _NTILES * _STRIPE, _TAIL)])
            plsc.subcore_barrier()

    return sc_kernel


def _prep_src(src, C):
    srcb = (src[None, :] + (jnp.arange(C, dtype=jnp.int32) * N)[:, None]
            ).reshape(C, _NB, _B)
    pad = jnp.zeros((C, 32 * _NBW - _NB, _B), jnp.int32)
    return (jnp.concatenate([srcb, pad], axis=1)
            .reshape(C, _NBW, 32, _B).transpose(0, 2, 1, 3)
            .reshape(C * 32 * _NBW, _B))


def _prep_dst(dst):
    dstb = dst.reshape(_NB, _B)
    pad = jnp.zeros((32 * _NBW - _NB, _B), jnp.int32)
    return (jnp.concatenate([dstb, pad], axis=0)
            .reshape(_NBW, 32, _B).transpose(1, 0, 2)
            .reshape(32 * _NBW, _B))


def _sc_segsum(hch, srcp, dstp, C):
    return _make_sc_segsum(C)(hch.reshape(C * N, 128), srcp, dstp
                              ).reshape(2, C, N, 128)


def _bf16_dot(a, b):
    return lax.dot_general(a.astype(jnp.bfloat16), b.astype(jnp.bfloat16),
                           (((1,), (0,)), ((), ())),
                           preferred_element_type=jnp.float32)


def _combine_body(C, chunked_out, hch_ref, zp_ref, wc_ref, bc_ref, out_ref):
    t = jnp.concatenate(
        [hch_ref[c] + zp_ref[0, c] + zp_ref[1, c] for c in range(C)], axis=-1)
    h = jnp.maximum(_bf16_dot(t, wc_ref[...]) + bc_ref[...], 0.0)
    if chunked_out:
        out_ref[...] = h.reshape(_R, 4, 128).transpose(1, 0, 2)
    else:
        out_ref[...] = h


def _combine(hch, zp, wc, bc, chunked_out):
    C = hch.shape[0]
    out_spec = (pl.BlockSpec((4, _R, 128), lambda i: (0, i, 0)) if chunked_out
                else pl.BlockSpec((_R, 512), lambda i: (i, 0)))
    out_shape = (jax.ShapeDtypeStruct((4, N, 128), jnp.float32) if chunked_out
                 else jax.ShapeDtypeStruct((N, 512), jnp.float32))
    return pl.pallas_call(
        functools.partial(_combine_body, C, chunked_out),
        grid=(_NR,),
        in_specs=[
            pl.BlockSpec((C, _R, 128), lambda i: (0, i, 0)),
            pl.BlockSpec((2, C, _R, 128), lambda i: (0, 0, i, 0)),
            pl.BlockSpec((C * 128, 512), lambda i: (0, 0)),
            pl.BlockSpec((1, 512), lambda i: (0, 0)),
        ],
        out_specs=out_spec,
        out_shape=out_shape,
    )(hch, zp, wc, bc)


def _readout_body(chunked, h_ref, wl_ref, bl_ref, n2g_ref, gf_ref):
    i = pl.program_id(0)
    if chunked:
        h = jnp.concatenate([h_ref[c] for c in range(4)], axis=-1)
    else:
        h = h_ref[...]
    logits = _bf16_dot(h, wl_ref[...]) + bl_ref[...]
    m = jnp.max(logits, axis=-1, keepdims=True)
    e = jnp.exp(logits - m)
    p = e / jnp.sum(e, axis=-1, keepdims=True)
    gids = lax.broadcasted_iota(jnp.int32, (G, _R), 0)
    sel = (gids == n2g_ref[0]).astype(jnp.float32)
    contrib = lax.dot_general(sel, p, (((1,), (0,)), ((), ())),
                              precision=_HI,
                              preferred_element_type=jnp.float32)

    @pl.when(i == 0)
    def _():
        gf_ref[...] = jnp.zeros_like(gf_ref)

    gf_ref[...] += contrib


def _readout(h, wl, bl, n2g3):
    chunked = h.ndim == 3
    h_spec = (pl.BlockSpec((4, _R, 128), lambda i: (0, i, 0)) if chunked
              else pl.BlockSpec((_R, 512), lambda i: (i, 0)))
    return pl.pallas_call(
        functools.partial(_readout_body, chunked),
        grid=(_NR,),
        in_specs=[
            h_spec,
            pl.BlockSpec((512, OUT), lambda i: (0, 0)),
            pl.BlockSpec((1, OUT), lambda i: (0, 0)),
            pl.BlockSpec((1, 1, _R), lambda i: (i, 0, 0)),
        ],
        out_specs=pl.BlockSpec((G, OUT), lambda i: (0, 0)),
        out_shape=jax.ShapeDtypeStruct((G, OUT), jnp.float32),
    )(h, wl, bl, n2g3)


def kernel(x, edge_index, node2graph, Wc0, bc0, Wl0, bl0, Wc1, bc1, Wl1, bl1,
           Wc2, bc2, Wl2, bl2):
    src = edge_index[0]
    dst = edge_index[1]
    src2 = _prep_src(src, 2)
    src4 = _prep_src(src, 4)
    dstp = _prep_dst(dst)
    n2g3 = node2graph.reshape(_NR, 1, _R)

    xch = x.reshape(N, 2, 128).transpose(1, 0, 2)

    z0 = _sc_segsum(xch, src2, dstp, 2)
    h0 = _combine(xch, z0, Wc0, bc0.reshape(1, 512), chunked_out=True)

    z1 = _sc_segsum(h0, src4, dstp, 4)
    gf0 = _readout(h0, Wl0, bl0.reshape(1, OUT), n2g3)

    h1 = _combine(h0, z1, Wc1, bc1.reshape(1, 512), chunked_out=True)

    z2 = _sc_segsum(h1, src4, dstp, 4)
    gf1 = _readout(h1, Wl1, bl1.reshape(1, OUT), n2g3)

    h2 = _combine(h1, z2, Wc2, bc2.reshape(1, 512), chunked_out=False)
    gf2 = _readout(h2, Wl2, bl2.reshape(1, OUT), n2g3)

    return (gf0 + gf1 + gf2, h2)


# speedup vs baseline: 1.2208x; 1.2208x over previous
import functools

import jax
import jax.numpy as jnp
from jax import lax
from jax.experimental import pallas as pl
from jax.experimental.pallas import tpu as pltpu
from jax.experimental.pallas import tpu_sc as plsc

N = 10000
E = 160000
G = 64
OUT = 2048

_B = 128
_NB = E // _B
_NBW = 40
_NTILES = 16
_STRIPE = 624
_TAIL = N - _NTILES * _STRIPE
_R = 400
_NR = N // _R

_HI = jax.lax.Precision.HIGHEST


@functools.lru_cache(maxsize=None)
def _make_sc_segsum(C):
    mesh = plsc.VectorSubcoreMesh(core_axis_name="c", subcore_axis_name="s",
                                  num_cores=2, num_subcores=_NTILES)

    @functools.partial(
        pl.kernel,
        out_type=jax.ShapeDtypeStruct((2 * C * N, 128), jnp.float32),
        mesh=mesh,
        scratch_types=[
            pltpu.VMEM((_NBW, _B), jnp.int32),
            pltpu.VMEM((_NBW, _B), jnp.int32),
            pltpu.VMEM((_B, 128), jnp.float32),
            pltpu.VMEM((_B, 128), jnp.float32),
            pltpu.VMEM((24, 128), jnp.float32),
            pltpu.VMEM_SHARED((N, 128), jnp.float32),
            pltpu.SemaphoreType.DMA,
            pltpu.SemaphoreType.DMA,
        ],
    )
    def sc_kernel(ych, src_hbm, dst_hbm, out_hbm, sidx, didx, rows0, rows1,
                  zbuf, table, sem0, sem1):
        cid = lax.axis_index("c")
        sid = lax.axis_index("s")
        w = cid * _NTILES + sid
        nb = jnp.where(w < _NB - (_NBW - 1) * 32, _NBW, _NBW - 1)

        @pl.loop(0, 24)
        def _(r):
            @pl.loop(0, 128, step=16)
            def _(j):
                zbuf.at[r, pl.ds(j, 16)][...] = jnp.zeros((16,), jnp.float32)

        pltpu.sync_copy(dst_hbm.at[pl.ds(w * _NBW, _NBW)], didx)

        for c in range(C):
            @pl.loop(0, _STRIPE // 24)
            def _(q):
                pltpu.sync_copy(
                    zbuf.at[pl.ds(0, 24)],
                    table.at[pl.ds(sid * _STRIPE + q * 24, 24)])

            @pl.when(sid == _NTILES - 1)
            def _():
                pltpu.sync_copy(zbuf.at[pl.ds(0, _TAIL)],
                                table.at[pl.ds(_NTILES * _STRIPE, _TAIL)])

            pltpu.sync_copy(src_hbm.at[pl.ds((c * 32 + w) * _NBW, _NBW)],
                            sidx)
            plsc.subcore_barrier()

            pltpu.async_copy(ych.at[sidx.at[0]], rows0, sem0)

            @pl.loop(0, _NBW // 2)
            def _(p):
                i0 = 2 * p
                i1 = i0 + 1

                @pl.when(i1 < nb)
                def _():
                    pltpu.async_copy(ych.at[sidx.at[i1]], rows1, sem1)
                pltpu.make_async_copy(ych.at[sidx.at[i0]], rows0, sem0).wait()
                pltpu.sync_copy(rows0, table.at[didx.at[i0]], add=True)

                @pl.when(i1 < nb)
                def _():
                    pltpu.make_async_copy(ych.at[sidx.at[i1]], rows1,
                                          sem1).wait()

                    @pl.when(i1 + 1 < nb)
                    def _():
                        pltpu.async_copy(ych.at[sidx.at[i1 + 1]], rows0, sem0)
                    pltpu.sync_copy(rows1, table.at[didx.at[i1]], add=True)
            plsc.subcore_barrier()

            base = (cid * C + c) * N
            pltpu.sync_copy(table.at[pl.ds(sid * _STRIPE, _STRIPE)],
                            out_hbm.at[pl.ds(base + sid * _STRIPE, _STRIPE)])

            @pl.when(sid == _NTILES - 1)
            def _():
                pltpu.sync_copy(
                    table.at[pl.ds(_NTILES * _STRIPE, _TAIL)],
                    out_hbm.at[pl.ds(base + _NTILES * _STRIPE, _TAIL)])
            plsc.subcore_barrier()

    return sc_kernel


def _prep_src(src, C):
    srcb = (src[None, :] + (jnp.arange(C, dtype=jnp.int32) * N)[:, None]
            ).reshape(C, _NB, _B)
    pad = jnp.zeros((C, 32 * _NBW - _NB, _B), jnp.int32)
    return (jnp.concatenate([srcb, pad], axis=1)
            .reshape(C, _NBW, 32, _B).transpose(0, 2, 1, 3)
            .reshape(C * 32 * _NBW, _B))


def _prep_dst(dst):
    dstb = dst.reshape(_NB, _B)
    pad = jnp.zeros((32 * _NBW - _NB, _B), jnp.int32)
    return (jnp.concatenate([dstb, pad], axis=0)
            .reshape(_NBW, 32, _B).transpose(1, 0, 2)
            .reshape(32 * _NBW, _B))


def _sc_segsum(hch, srcp, dstp, C):
    return _make_sc_segsum(C)(hch.reshape(C * N, 128), srcp, dstp
                              ).reshape(2, C, N, 128)


def _bf16_dot(a, b):
    return lax.dot_general(a.astype(jnp.bfloat16), b.astype(jnp.bfloat16),
                           (((1,), (0,)), ((), ())),
                           preferred_element_type=jnp.float32)


def _combine_body(C, chunked_out, hch_ref, zp_ref, wc_ref, bc_ref, out_ref):
    t = jnp.concatenate(
        [hch_ref[c] + zp_ref[0, c] + zp_ref[1, c] for c in range(C)], axis=-1)
    h = jnp.maximum(_bf16_dot(t, wc_ref[...]) + bc_ref[...], 0.0)
    if chunked_out:
        out_ref[...] = h.reshape(_R, 4, 128).transpose(1, 0, 2)
    else:
        out_ref[...] = h


def _combine(hch, zp, wc, bc, chunked_out):
    C = hch.shape[0]
    out_spec = (pl.BlockSpec((4, _R, 128), lambda i: (0, i, 0)) if chunked_out
                else pl.BlockSpec((_R, 512), lambda i: (i, 0)))
    out_shape = (jax.ShapeDtypeStruct((4, N, 128), jnp.float32) if chunked_out
                 else jax.ShapeDtypeStruct((N, 512), jnp.float32))
    return pl.pallas_call(
        functools.partial(_combine_body, C, chunked_out),
        grid=(_NR,),
        in_specs=[
            pl.BlockSpec((C, _R, 128), lambda i: (0, i, 0)),
            pl.BlockSpec((2, C, _R, 128), lambda i: (0, 0, i, 0)),
            pl.BlockSpec((C * 128, 512), lambda i: (0, 0)),
            pl.BlockSpec((1, 512), lambda i: (0, 0)),
        ],
        out_specs=out_spec,
        out_shape=out_shape,
    )(hch, zp, wc, bc)


def _combine_readout_body(hch_ref, zp_ref, wc_ref, bc_ref, wl_ref, bl_ref,
                          n2g_ref, h_ref, gf_ref):
    i = pl.program_id(0)
    t = jnp.concatenate(
        [hch_ref[c] + zp_ref[0, c] + zp_ref[1, c] for c in range(4)], axis=-1)
    h = jnp.maximum(_bf16_dot(t, wc_ref[...]) + bc_ref[...], 0.0)
    h_ref[...] = h
    logits = _bf16_dot(h, wl_ref[...]) + bl_ref[...]
    m = jnp.max(logits, axis=-1, keepdims=True)
    e = jnp.exp(logits - m)
    p = e / jnp.sum(e, axis=-1, keepdims=True)
    gids = lax.broadcasted_iota(jnp.int32, (G, _R), 0)
    sel = (gids == n2g_ref[0]).astype(jnp.float32)
    contrib = lax.dot_general(sel, p, (((1,), (0,)), ((), ())),
                              precision=_HI,
                              preferred_element_type=jnp.float32)

    @pl.when(i == 0)
    def _():
        gf_ref[...] = jnp.zeros_like(gf_ref)

    gf_ref[...] += contrib


def _combine_readout(hch, zp, wc, bc, wl, bl, n2g3):
    return pl.pallas_call(
        _combine_readout_body,
        grid=(_NR,),
        in_specs=[
            pl.BlockSpec((4, _R, 128), lambda i: (0, i, 0)),
            pl.BlockSpec((2, 4, _R, 128), lambda i: (0, 0, i, 0)),
            pl.BlockSpec((512, 512), lambda i: (0, 0)),
            pl.BlockSpec((1, 512), lambda i: (0, 0)),
            pl.BlockSpec((512, OUT), lambda i: (0, 0)),
            pl.BlockSpec((1, OUT), lambda i: (0, 0)),
            pl.BlockSpec((1, 1, _R), lambda i: (i, 0, 0)),
        ],
        out_specs=[
            pl.BlockSpec((_R, 512), lambda i: (i, 0)),
            pl.BlockSpec((G, OUT), lambda i: (0, 0)),
        ],
        out_shape=[
            jax.ShapeDtypeStruct((N, 512), jnp.float32),
            jax.ShapeDtypeStruct((G, OUT), jnp.float32),
        ],
    )(hch, zp, wc, bc, wl, bl, n2g3)


def _readout_body(chunked, h_ref, wl_ref, bl_ref, n2g_ref, gf_ref):
    i = pl.program_id(0)
    if chunked:
        h = jnp.concatenate([h_ref[c] for c in range(4)], axis=-1)
    else:
        h = h_ref[...]
    logits = _bf16_dot(h, wl_ref[...]) + bl_ref[...]
    m = jnp.max(logits, axis=-1, keepdims=True)
    e = jnp.exp(logits - m)
    p = e / jnp.sum(e, axis=-1, keepdims=True)
    gids = lax.broadcasted_iota(jnp.int32, (G, _R), 0)
    sel = (gids == n2g_ref[0]).astype(jnp.float32)
    contrib = lax.dot_general(sel, p, (((1,), (0,)), ((), ())),
                              precision=_HI,
                              preferred_element_type=jnp.float32)

    @pl.when(i == 0)
    def _():
        gf_ref[...] = jnp.zeros_like(gf_ref)

    gf_ref[...] += contrib


def _readout(h, wl, bl, n2g3):
    chunked = h.ndim == 3
    h_spec = (pl.BlockSpec((4, _R, 128), lambda i: (0, i, 0)) if chunked
              else pl.BlockSpec((_R, 512), lambda i: (i, 0)))
    return pl.pallas_call(
        functools.partial(_readout_body, chunked),
        grid=(_NR,),
        in_specs=[
            h_spec,
            pl.BlockSpec((512, OUT), lambda i: (0, 0)),
            pl.BlockSpec((1, OUT), lambda i: (0, 0)),
            pl.BlockSpec((1, 1, _R), lambda i: (i, 0, 0)),
        ],
        out_specs=pl.BlockSpec((G, OUT), lambda i: (0, 0)),
        out_shape=jax.ShapeDtypeStruct((G, OUT), jnp.float32),
    )(h, wl, bl, n2g3)


def kernel(x, edge_index, node2graph, Wc0, bc0, Wl0, bl0, Wc1, bc1, Wl1, bl1,
           Wc2, bc2, Wl2, bl2):
    src = edge_index[0]
    dst = edge_index[1]
    src2 = _prep_src(src, 2)
    src4 = _prep_src(src, 4)
    dstp = _prep_dst(dst)
    n2g3 = node2graph.reshape(_NR, 1, _R)

    xch = x.reshape(N, 2, 128).transpose(1, 0, 2)

    z0 = _sc_segsum(xch, src2, dstp, 2)
    h0 = _combine(xch, z0, Wc0, bc0.reshape(1, 512), chunked_out=True)

    z1 = _sc_segsum(h0, src4, dstp, 4)
    gf0 = _readout(h0, Wl0, bl0.reshape(1, OUT), n2g3)

    h1 = _combine(h0, z1, Wc1, bc1.reshape(1, 512), chunked_out=True)

    z2 = _sc_segsum(h1, src4, dstp, 4)
    gf1 = _readout(h1, Wl1, bl1.reshape(1, OUT), n2g3)

    h2, gf2 = _combine_readout(h1, z2, Wc2, bc2.reshape(1, 512), Wl2,
                               bl2.reshape(1, OUT), n2g3)

    return (gf0 + gf1 + gf2, h2)


# speedup vs baseline: 1.2486x; 1.0228x over previous
import functools

import jax
import jax.numpy as jnp
from jax import lax
from jax.experimental import pallas as pl
from jax.experimental.pallas import tpu as pltpu
from jax.experimental.pallas import tpu_sc as plsc

N = 10000
E = 160000
G = 64
OUT = 2048

_B = 128
_NB = E // _B
_NBW = 40
_NTILES = 16
_STRIPE = 624
_TAIL = N - _NTILES * _STRIPE
_R = 1000
_NR = N // _R

_HI = jax.lax.Precision.HIGHEST


@functools.lru_cache(maxsize=None)
def _make_sc_segsum(C):
    mesh = plsc.VectorSubcoreMesh(core_axis_name="c", subcore_axis_name="s",
                                  num_cores=2, num_subcores=_NTILES)

    @functools.partial(
        pl.kernel,
        out_type=jax.ShapeDtypeStruct((2 * C * N, 128), jnp.float32),
        mesh=mesh,
        scratch_types=[
            pltpu.VMEM((_NBW, _B), jnp.int32),
            pltpu.VMEM((_NBW, _B), jnp.int32),
            pltpu.VMEM((_B, 128), jnp.float32),
            pltpu.VMEM((_B, 128), jnp.float32),
            pltpu.VMEM((24, 128), jnp.float32),
            pltpu.VMEM_SHARED((N, 128), jnp.float32),
            pltpu.SemaphoreType.DMA,
            pltpu.SemaphoreType.DMA,
        ],
    )
    def sc_kernel(ych, src_hbm, dst_hbm, out_hbm, sidx, didx, rows0, rows1,
                  zbuf, table, sem0, sem1):
        cid = lax.axis_index("c")
        sid = lax.axis_index("s")
        w = cid * _NTILES + sid
        nb = jnp.where(w < _NB - (_NBW - 1) * 32, _NBW, _NBW - 1)

        @pl.loop(0, 24)
        def _(r):
            @pl.loop(0, 128, step=16)
            def _(j):
                zbuf.at[r, pl.ds(j, 16)][...] = jnp.zeros((16,), jnp.float32)

        pltpu.sync_copy(dst_hbm.at[pl.ds(w * _NBW, _NBW)], didx)

        for c in range(C):
            @pl.loop(0, _STRIPE // 24)
            def _(q):
                pltpu.sync_copy(
                    zbuf.at[pl.ds(0, 24)],
                    table.at[pl.ds(sid * _STRIPE + q * 24, 24)])

            @pl.when(sid == _NTILES - 1)
            def _():
                pltpu.sync_copy(zbuf.at[pl.ds(0, _TAIL)],
                                table.at[pl.ds(_NTILES * _STRIPE, _TAIL)])

            pltpu.sync_copy(src_hbm.at[pl.ds((c * 32 + w) * _NBW, _NBW)],
                            sidx)
            plsc.subcore_barrier()

            pltpu.async_copy(ych.at[sidx.at[0]], rows0, sem0)

            @pl.loop(0, _NBW // 2)
            def _(p):
                i0 = 2 * p
                i1 = i0 + 1

                @pl.when(i1 < nb)
                def _():
                    pltpu.async_copy(ych.at[sidx.at[i1]], rows1, sem1)
                pltpu.make_async_copy(ych.at[sidx.at[i0]], rows0, sem0).wait()
                pltpu.sync_copy(rows0, table.at[didx.at[i0]], add=True)

                @pl.when(i1 < nb)
                def _():
                    pltpu.make_async_copy(ych.at[sidx.at[i1]], rows1,
                                          sem1).wait()

                    @pl.when(i1 + 1 < nb)
                    def _():
                        pltpu.async_copy(ych.at[sidx.at[i1 + 1]], rows0, sem0)
                    pltpu.sync_copy(rows1, table.at[didx.at[i1]], add=True)
            plsc.subcore_barrier()

            base = (cid * C + c) * N
            pltpu.sync_copy(table.at[pl.ds(sid * _STRIPE, _STRIPE)],
                            out_hbm.at[pl.ds(base + sid * _STRIPE, _STRIPE)])

            @pl.when(sid == _NTILES - 1)
            def _():
                pltpu.sync_copy(
                    table.at[pl.ds(_NTILES * _STRIPE, _TAIL)],
                    out_hbm.at[pl.ds(base + _NTILES * _STRIPE, _TAIL)])
            plsc.subcore_barrier()

    return sc_kernel


def _prep_src(src, C):
    srcb = (src[None, :] + (jnp.arange(C, dtype=jnp.int32) * N)[:, None]
            ).reshape(C, _NB, _B)
    pad = jnp.zeros((C, 32 * _NBW - _NB, _B), jnp.int32)
    return (jnp.concatenate([srcb, pad], axis=1)
            .reshape(C, _NBW, 32, _B).transpose(0, 2, 1, 3)
            .reshape(C * 32 * _NBW, _B))


def _prep_dst(dst):
    dstb = dst.reshape(_NB, _B)
    pad = jnp.zeros((32 * _NBW - _NB, _B), jnp.int32)
    return (jnp.concatenate([dstb, pad], axis=0)
            .reshape(_NBW, 32, _B).transpose(1, 0, 2)
            .reshape(32 * _NBW, _B))


def _sc_segsum(hch, srcp, dstp, C):
    return _make_sc_segsum(C)(hch.reshape(C * N, 128), srcp, dstp
                              ).reshape(2, C, N, 128)


def _bf16_dot(a, b):
    return lax.dot_general(a.astype(jnp.bfloat16), b.astype(jnp.bfloat16),
                           (((1,), (0,)), ((), ())),
                           preferred_element_type=jnp.float32)


def _combine_body(C, chunked_out, hch_ref, zp_ref, wc_ref, bc_ref, out_ref):
    t = jnp.concatenate(
        [hch_ref[c] + zp_ref[0, c] + zp_ref[1, c] for c in range(C)], axis=-1)
    h = jnp.maximum(_bf16_dot(t, wc_ref[...]) + bc_ref[...], 0.0)
    if chunked_out:
        out_ref[...] = h.reshape(_R, 4, 128).transpose(1, 0, 2)
    else:
        out_ref[...] = h


def _combine(hch, zp, wc, bc, chunked_out):
    C = hch.shape[0]
    out_spec = (pl.BlockSpec((4, _R, 128), lambda i: (0, i, 0)) if chunked_out
                else pl.BlockSpec((_R, 512), lambda i: (i, 0)))
    out_shape = (jax.ShapeDtypeStruct((4, N, 128), jnp.float32) if chunked_out
                 else jax.ShapeDtypeStruct((N, 512), jnp.float32))
    return pl.pallas_call(
        functools.partial(_combine_body, C, chunked_out),
        grid=(_NR,),
        in_specs=[
            pl.BlockSpec((C, _R, 128), lambda i: (0, i, 0)),
            pl.BlockSpec((2, C, _R, 128), lambda i: (0, 0, i, 0)),
            pl.BlockSpec((C * 128, 512), lambda i: (0, 0)),
            pl.BlockSpec((1, 512), lambda i: (0, 0)),
        ],
        out_specs=out_spec,
        out_shape=out_shape,
    )(hch, zp, wc, bc)


def _combine_readout_body(hch_ref, zp_ref, wc_ref, bc_ref, wl_ref, bl_ref,
                          n2g_ref, h_ref, gf_ref):
    i = pl.program_id(0)
    t = jnp.concatenate(
        [hch_ref[c] + zp_ref[0, c] + zp_ref[1, c] for c in range(4)], axis=-1)
    h = jnp.maximum(_bf16_dot(t, wc_ref[...]) + bc_ref[...], 0.0)
    h_ref[...] = h
    logits = _bf16_dot(h, wl_ref[...]) + bl_ref[...]
    m = jnp.max(logits, axis=-1, keepdims=True)
    e = jnp.exp(logits - m)
    p = e / jnp.sum(e, axis=-1, keepdims=True)
    gids = lax.broadcasted_iota(jnp.int32, (G, _R), 0)
    sel = (gids == n2g_ref[0]).astype(jnp.float32)
    contrib = lax.dot_general(sel, p, (((1,), (0,)), ((), ())),
                              precision=_HI,
                              preferred_element_type=jnp.float32)

    @pl.when(i == 0)
    def _():
        gf_ref[...] = jnp.zeros_like(gf_ref)

    gf_ref[...] += contrib


def _combine_readout(hch, zp, wc, bc, wl, bl, n2g3):
    return pl.pallas_call(
        _combine_readout_body,
        grid=(_NR,),
        in_specs=[
            pl.BlockSpec((4, _R, 128), lambda i: (0, i, 0)),
            pl.BlockSpec((2, 4, _R, 128), lambda i: (0, 0, i, 0)),
            pl.BlockSpec((512, 512), lambda i: (0, 0)),
            pl.BlockSpec((1, 512), lambda i: (0, 0)),
            pl.BlockSpec((512, OUT), lambda i: (0, 0)),
            pl.BlockSpec((1, OUT), lambda i: (0, 0)),
            pl.BlockSpec((1, 1, _R), lambda i: (i, 0, 0)),
        ],
        out_specs=[
            pl.BlockSpec((_R, 512), lambda i: (i, 0)),
            pl.BlockSpec((G, OUT), lambda i: (0, 0)),
        ],
        out_shape=[
            jax.ShapeDtypeStruct((N, 512), jnp.float32),
            jax.ShapeDtypeStruct((G, OUT), jnp.float32),
        ],
    )(hch, zp, wc, bc, wl, bl, n2g3)


def _readout_body(chunked, h_ref, wl_ref, bl_ref, n2g_ref, gf_ref):
    i = pl.program_id(0)
    if chunked:
        h = jnp.concatenate([h_ref[c] for c in range(4)], axis=-1)
    else:
        h = h_ref[...]
    logits = _bf16_dot(h, wl_ref[...]) + bl_ref[...]
    m = jnp.max(logits, axis=-1, keepdims=True)
    e = jnp.exp(logits - m)
    p = e / jnp.sum(e, axis=-1, keepdims=True)
    gids = lax.broadcasted_iota(jnp.int32, (G, _R), 0)
    sel = (gids == n2g_ref[0]).astype(jnp.float32)
    contrib = lax.dot_general(sel, p, (((1,), (0,)), ((), ())),
                              precision=_HI,
                              preferred_element_type=jnp.float32)

    @pl.when(i == 0)
    def _():
        gf_ref[...] = jnp.zeros_like(gf_ref)

    gf_ref[...] += contrib


def _readout(h, wl, bl, n2g3):
    chunked = h.ndim == 3
    h_spec = (pl.BlockSpec((4, _R, 128), lambda i: (0, i, 0)) if chunked
              else pl.BlockSpec((_R, 512), lambda i: (i, 0)))
    return pl.pallas_call(
        functools.partial(_readout_body, chunked),
        grid=(_NR,),
        in_specs=[
            h_spec,
            pl.BlockSpec((512, OUT), lambda i: (0, 0)),
            pl.BlockSpec((1, OUT), lambda i: (0, 0)),
            pl.BlockSpec((1, 1, _R), lambda i: (i, 0, 0)),
        ],
        out_specs=pl.BlockSpec((G, OUT), lambda i: (0, 0)),
        out_shape=jax.ShapeDtypeStruct((G, OUT), jnp.float32),
    )(h, wl, bl, n2g3)


def kernel(x, edge_index, node2graph, Wc0, bc0, Wl0, bl0, Wc1, bc1, Wl1, bl1,
           Wc2, bc2, Wl2, bl2):
    src = edge_index[0]
    dst = edge_index[1]
    src2 = _prep_src(src, 2)
    src4 = _prep_src(src, 4)
    dstp = _prep_dst(dst)
    n2g3 = node2graph.reshape(_NR, 1, _R)

    xch = x.reshape(N, 2, 128).transpose(1, 0, 2)

    z0 = _sc_segsum(xch, src2, dstp, 2)
    h0 = _combine(xch, z0, Wc0, bc0.reshape(1, 512), chunked_out=True)

    z1 = _sc_segsum(h0, src4, dstp, 4)
    gf0 = _readout(h0, Wl0, bl0.reshape(1, OUT), n2g3)

    h1 = _combine(h0, z1, Wc1, bc1.reshape(1, 512), chunked_out=True)

    z2 = _sc_segsum(h1, src4, dstp, 4)
    gf1 = _readout(h1, Wl1, bl1.reshape(1, OUT), n2g3)

    h2, gf2 = _combine_readout(h1, z2, Wc2, bc2.reshape(1, 512), Wl2,
                               bl2.reshape(1, OUT), n2g3)

    return (gf0 + gf1 + gf2, h2)
